# baseline (speedup 1.0000x reference)
import jax
import jax.numpy as jnp
from jax import lax
from jax.experimental import pallas as pl
from jax.experimental.pallas import tpu as pltpu
from jax.experimental.pallas import tpu_sc as plsc

N = 10000
NPAD = 10240
RPT = NPAD // 16
D = 128
H = 64
G = 64
E = 320000
CH = 125
NTILES = 32
NCH = 80
_HI = lax.Precision.HIGHEST


def _sc_mesh():
    return plsc.VectorSubcoreMesh(core_axis_name="c", subcore_axis_name="s")


def _deg_pass(ei, zeros8, ones8):
    def body(ei_hbm, z_hbm, ones_hbm, out_hbm, dst_v, ones_v, acc, t0, t1,
             t2):
        cid = lax.axis_index("c")
        sid = lax.axis_index("s")
        wid = cid * 16 + sid
        r0 = sid * RPT
        c0 = pltpu.async_copy(z_hbm.at[pl.ds(r0, RPT)],
                              acc.at[pl.ds(r0, RPT)], t0)
        c1 = pltpu.async_copy(ones_hbm, ones_v, t1)
        c2 = pltpu.async_copy(ei_hbm.at[1, wid], dst_v, t2)
        c0.wait()
        c1.wait()
        c2.wait()
        plsc.subcore_barrier()

        def chunk(j, c):
            pltpu.sync_copy(ones_v, acc.at[dst_v.at[j]], add=True)
            return c

        lax.fori_loop(0, NCH, chunk, 0)
        plsc.subcore_barrier()
        pltpu.sync_copy(acc.at[pl.ds(r0, RPT)],
                        out_hbm.at[pl.ds(cid * NPAD + r0, RPT)])

    f = pl.kernel(
        body,
        out_type=jax.ShapeDtypeStruct((2 * NPAD, 8), jnp.bfloat16),
        mesh=_sc_mesh(),
        compiler_params=pltpu.CompilerParams(use_tc_tiling_on_sc=False),
        scratch_types=[
            pltpu.VMEM((NCH, CH), jnp.int32),
            pltpu.VMEM((CH, 8), jnp.bfloat16),
            pltpu.VMEM_SHARED((NPAD, 8), jnp.bfloat16),
            pltpu.SemaphoreType.DMA,
            pltpu.SemaphoreType.DMA,
            pltpu.SemaphoreType.DMA,
        ],
    )
    return f(ei, zeros8, ones8)


def _agg_pass(y, zeros64, ei):
    NB = 4
    NG = NCH // NB

    def body(y_hbm, z_hbm, ei_hbm, out_hbm, src_v, dst_v,
             bufs, s0, s1, s2, s3, s4, s5, s6, s7, acc, ytab):
        cid = lax.axis_index("c")
        sid = lax.axis_index("s")
        wid = cid * 16 + sid
        r0 = sid * RPT
        sems = (s0, s1, s2, s3, s4, s5, s6, s7)

        pltpu.sync_copy(z_hbm.at[pl.ds(r0, RPT)], acc.at[pl.ds(r0, RPT)])
        pltpu.sync_copy(y_hbm.at[pl.ds(r0, RPT)], ytab.at[pl.ds(r0, RPT)])
        pltpu.sync_copy(ei_hbm.at[0, wid], src_v)
        pltpu.sync_copy(ei_hbm.at[1, wid], dst_v)
        plsc.subcore_barrier()
        for b in range(NB):
            pltpu.async_copy(ytab.at[src_v.at[b]], bufs.at[b], sems[b])


        def group(g, c):
            def half(par):
                lo = par * NB
                pf = NB - lo
                nxt = (g + 1) * NB

                @pl.when(g + 1 < NG)
                def _():
                    for b in range(NB):
                        pltpu.async_copy(ytab.at[src_v.at[nxt + b]],
                                         bufs.at[pf + b], sems[pf + b])

                for b in range(NB):
                    j = g * NB + b
                    pltpu.make_async_copy(ytab.at[src_v.at[j]],
                                          bufs.at[lo + b], sems[lo + b]).wait()
                    pltpu.sync_copy(bufs.at[lo + b], acc.at[dst_v.at[j]],
                                    add=True)

            @pl.when(g % 2 == 0)
            def _():
                half(0)

            @pl.when(g % 2 == 1)
            def _():
                half(1)

            return c

        lax.fori_loop(0, NG, group, 0)
        plsc.subcore_barrier()
        pltpu.sync_copy(acc.at[pl.ds(r0, RPT)],
                        out_hbm.at[pl.ds(cid * NPAD + r0, RPT)])

    f = pl.kernel(
        body,
        out_type=jax.ShapeDtypeStruct((2 * NPAD, H), jnp.bfloat16),
        mesh=_sc_mesh(),
        compiler_params=pltpu.CompilerParams(use_tc_tiling_on_sc=False),
        scratch_types=[
            pltpu.VMEM((NCH, CH), jnp.int32),
            pltpu.VMEM((NCH, CH), jnp.int32),
            pltpu.VMEM((8, CH, H), jnp.bfloat16),
            pltpu.SemaphoreType.DMA,
            pltpu.SemaphoreType.DMA,
            pltpu.SemaphoreType.DMA,
            pltpu.SemaphoreType.DMA,
            pltpu.SemaphoreType.DMA,
            pltpu.SemaphoreType.DMA,
            pltpu.SemaphoreType.DMA,
            pltpu.SemaphoreType.DMA,
            pltpu.VMEM_SHARED((NPAD, H), jnp.bfloat16),
            pltpu.VMEM_SHARED((NPAD, H), jnp.bfloat16),
        ],
    )
    return f(y, zeros64, ei)


def _k1(x, W1, degp):
    def body(x_ref, w_ref, degp_ref, y_ref, dinv_ref):
        deg = (degp_ref[0:NPAD, 0:1].astype(jnp.float32)
               + degp_ref[NPAD:2 * NPAD, 0:1].astype(jnp.float32) + 1.0)
        rows = lax.broadcasted_iota(jnp.int32, (NPAD, 1), 0)
        dinv = jnp.where(rows < N, lax.rsqrt(deg), 0.0)
        xw = jnp.dot(
            x_ref[...].astype(jnp.bfloat16), w_ref[...].astype(jnp.bfloat16),
            preferred_element_type=jnp.float32)
        yv = xw * dinv[0:N]
        y_ref[...] = jnp.concatenate(
            [yv, jnp.zeros((NPAD - N, H), jnp.float32)], axis=0
        ).astype(jnp.bfloat16)
        dinv_ref[...] = jnp.broadcast_to(dinv, (NPAD, 8))

    return pl.pallas_call(
        body,
        out_shape=(jax.ShapeDtypeStruct((NPAD, H), jnp.bfloat16),
                   jax.ShapeDtypeStruct((NPAD, 8), jnp.float32)),
    )(x, W1, degp)


def _k23(a, y, dinv8, b, W):
    def body(a_ref, y_ref_in, dinv_ref, b_ref, w_ref, y_ref):
        dinv = dinv_ref[:, 0:1]
        s = (a_ref[0:NPAD, :].astype(jnp.float32)
             + a_ref[NPAD:2 * NPAD, :].astype(jnp.float32)
             + y_ref_in[...].astype(jnp.float32))
        h = jnp.maximum(dinv * s + b_ref[...], 0.0)
        y_ref[...] = (dinv * jnp.dot(
            h.astype(jnp.bfloat16), w_ref[...].astype(jnp.bfloat16),
            preferred_element_type=jnp.float32)).astype(jnp.bfloat16)

    return pl.pallas_call(
        body,
        out_shape=jax.ShapeDtypeStruct((NPAD, H), jnp.bfloat16),
    )(a, y, dinv8, b, W)


def _k4(a, y, dinv8, b3, batchp, Wf1, bf1, Wf2, bf2):
    def body(a_ref, y_ref, dinv_ref, b_ref, batch_ref, wf1_ref,
             bf1_ref, wf2_ref, bf2_ref, out_ref):
        dinv = dinv_ref[:, 0:1]
        s = (a_ref[0:NPAD, :].astype(jnp.float32)
             + a_ref[NPAD:2 * NPAD, :].astype(jnp.float32)
             + y_ref[...].astype(jnp.float32))
        h = jnp.maximum(dinv * s + b_ref[...], 0.0)
        gids = lax.broadcasted_iota(jnp.int32, (G, NPAD), 0)
        mt = (gids == batch_ref[...]).astype(jnp.bfloat16)
        sums = jnp.dot(mt, h.astype(jnp.bfloat16),
                       preferred_element_type=jnp.float32)
        counts = jnp.sum(mt.astype(jnp.float32), axis=1, keepdims=True)
        pooled = sums / jnp.maximum(counts, 1.0)
        hh = jnp.maximum(jnp.dot(pooled, wf1_ref[...], precision=_HI,
                                 preferred_element_type=jnp.float32)
                         + bf1_ref[...], 0.0)
        logits = jnp.dot(hh, wf2_ref[...], precision=_HI,
                         preferred_element_type=jnp.float32) + bf2_ref[...]
        ls = logits - jnp.max(logits, axis=1, keepdims=True)
        out_ref[...] = ls - jnp.log(jnp.sum(jnp.exp(ls), axis=1,
                                            keepdims=True))

    return pl.pallas_call(
        body,
        out_shape=jax.ShapeDtypeStruct((G, 2), jnp.float32),
    )(a, y, dinv8, b3, batchp, Wf1, bf1, Wf2, bf2)


def kernel(x, edge_index, batch, W1, b1, W2, b2, W3, b3, Wf1, bf1, Wf2, bf2):
    f32 = jnp.float32
    ei = edge_index.reshape(2, NTILES, NCH, CH)
    batchp = jnp.concatenate(
        [batch, jnp.full((NPAD - N,), -1, jnp.int32)]).reshape(1, NPAD)
    zeros64 = jnp.zeros((NPAD, H), jnp.bfloat16)
    zeros8 = jnp.zeros((NPAD, 8), jnp.bfloat16)
    ones8 = jnp.ones((CH, 8), jnp.bfloat16)

    degp = _deg_pass(ei, zeros8, ones8)
    y, dinv8 = _k1(x, W1, degp)
    a = _agg_pass(y, zeros64, ei)
    y = _k23(a, y, dinv8, b1.reshape(1, H), W2)
    a = _agg_pass(y, zeros64, ei)
    y = _k23(a, y, dinv8, b2.reshape(1, H), W3)
    a = _agg_pass(y, zeros64, ei)
    return _k4(a, y, dinv8, b3.reshape(1, H), batchp,
               Wf1, bf1.reshape(1, 32), Wf2, bf2.reshape(1, 2))


# speedup vs baseline: 1.0259x; 1.0259x over previous
import jax
import jax.numpy as jnp
from jax import lax
from jax.experimental import pallas as pl
from jax.experimental.pallas import tpu as pltpu
from jax.experimental.pallas import tpu_sc as plsc

N = 10000
NPAD = 10240
RPT = NPAD // 16
D = 128
H = 64
G = 64
E = 320000
CH = 125
NTILES = 32
NCH = 80
_HI = lax.Precision.HIGHEST


def _sc_mesh():
    return plsc.VectorSubcoreMesh(core_axis_name="c", subcore_axis_name="s")


def _deg_pass(ei, zeros8, ones8):
    def body(ei_hbm, z_hbm, ones_hbm, out_hbm, dst_v, ones_v, acc, t0, t1,
             t2):
        cid = lax.axis_index("c")
        sid = lax.axis_index("s")
        wid = cid * 16 + sid
        r0 = sid * RPT
        c0 = pltpu.async_copy(z_hbm.at[pl.ds(r0, RPT)],
                              acc.at[pl.ds(r0, RPT)], t0)
        c1 = pltpu.async_copy(ones_hbm, ones_v, t1)
        c2 = pltpu.async_copy(ei_hbm.at[1, wid], dst_v, t2)
        c0.wait()
        c1.wait()
        c2.wait()
        plsc.subcore_barrier()

        def chunk(j, c):
            pltpu.sync_copy(ones_v, acc.at[dst_v.at[j]], add=True)
            return c

        lax.fori_loop(0, NCH, chunk, 0)
        plsc.subcore_barrier()
        pltpu.sync_copy(acc.at[pl.ds(r0, RPT)],
                        out_hbm.at[pl.ds(cid * NPAD + r0, RPT)])

    f = pl.kernel(
        body,
        out_type=jax.ShapeDtypeStruct((2 * NPAD, 8), jnp.bfloat16),
        mesh=_sc_mesh(),
        compiler_params=pltpu.CompilerParams(use_tc_tiling_on_sc=False),
        scratch_types=[
            pltpu.VMEM((NCH, CH), jnp.int32),
            pltpu.VMEM((CH, 8), jnp.bfloat16),
            pltpu.VMEM_SHARED((NPAD, 8), jnp.bfloat16),
            pltpu.SemaphoreType.DMA,
            pltpu.SemaphoreType.DMA,
            pltpu.SemaphoreType.DMA,
        ],
    )
    return f(ei, zeros8, ones8)


def _agg_pass(y, zeros64, ei):
    NB = 4
    NG = NCH // NB

    def body(y_hbm, z_hbm, ei_hbm, out_hbm, src_v, dst_v,
             bufs, s0, s1, s2, s3, s4, s5, s6, s7, acc, ytab):
        cid = lax.axis_index("c")
        sid = lax.axis_index("s")
        wid = cid * 16 + sid
        r0 = sid * RPT
        sems = (s0, s1, s2, s3, s4, s5, s6, s7)

        c2 = pltpu.async_copy(ei_hbm.at[0, wid], src_v, s4)
        c3 = pltpu.async_copy(ei_hbm.at[1, wid], dst_v, s5)
        pltpu.sync_copy(z_hbm.at[pl.ds(r0, RPT)], acc.at[pl.ds(r0, RPT)])
        pltpu.sync_copy(y_hbm.at[pl.ds(r0, RPT)], ytab.at[pl.ds(r0, RPT)])
        c2.wait()
        c3.wait()
        plsc.subcore_barrier()
        for b in range(NB):
            pltpu.async_copy(ytab.at[src_v.at[b]], bufs.at[b], sems[b])


        def group(g, c):
            def half(par):
                lo = par * NB
                pf = NB - lo
                nxt = (g + 1) * NB

                @pl.when(g + 1 < NG)
                def _():
                    for b in range(NB):
                        pltpu.async_copy(ytab.at[src_v.at[nxt + b]],
                                         bufs.at[pf + b], sems[pf + b])

                for b in range(NB):
                    j = g * NB + b
                    pltpu.make_async_copy(ytab.at[src_v.at[j]],
                                          bufs.at[lo + b], sems[lo + b]).wait()
                    pltpu.sync_copy(bufs.at[lo + b], acc.at[dst_v.at[j]],
                                    add=True)

            @pl.when(g % 2 == 0)
            def _():
                half(0)

            @pl.when(g % 2 == 1)
            def _():
                half(1)

            return c

        lax.fori_loop(0, NG, group, 0)
        plsc.subcore_barrier()
        pltpu.sync_copy(acc.at[pl.ds(r0, RPT)],
                        out_hbm.at[pl.ds(cid * NPAD + r0, RPT)])

    f = pl.kernel(
        body,
        out_type=jax.ShapeDtypeStruct((2 * NPAD, H), jnp.bfloat16),
        mesh=_sc_mesh(),
        compiler_params=pltpu.CompilerParams(use_tc_tiling_on_sc=False),
        scratch_types=[
            pltpu.VMEM((NCH, CH), jnp.int32),
            pltpu.VMEM((NCH, CH), jnp.int32),
            pltpu.VMEM((8, CH, H), jnp.bfloat16),
            pltpu.SemaphoreType.DMA,
            pltpu.SemaphoreType.DMA,
            pltpu.SemaphoreType.DMA,
            pltpu.SemaphoreType.DMA,
            pltpu.SemaphoreType.DMA,
            pltpu.SemaphoreType.DMA,
            pltpu.SemaphoreType.DMA,
            pltpu.SemaphoreType.DMA,
            pltpu.VMEM_SHARED((NPAD, H), jnp.bfloat16),
            pltpu.VMEM_SHARED((NPAD, H), jnp.bfloat16),
        ],
    )
    return f(y, zeros64, ei)


def _k1(x, W1, degp):
    def body(x_ref, w_ref, degp_ref, y_ref, dinv_ref):
        deg = (degp_ref[0:NPAD, 0:1].astype(jnp.float32)
               + degp_ref[NPAD:2 * NPAD, 0:1].astype(jnp.float32) + 1.0)
        rows = lax.broadcasted_iota(jnp.int32, (NPAD, 1), 0)
        dinv = jnp.where(rows < N, lax.rsqrt(deg), 0.0)
        xw = jnp.dot(
            x_ref[...].astype(jnp.bfloat16), w_ref[...].astype(jnp.bfloat16),
            preferred_element_type=jnp.float32)
        yv = xw * dinv[0:N]
        y_ref[...] = jnp.concatenate(
            [yv, jnp.zeros((NPAD - N, H), jnp.float32)], axis=0
        ).astype(jnp.bfloat16)
        dinv_ref[...] = jnp.broadcast_to(dinv, (NPAD, 8))

    return pl.pallas_call(
        body,
        out_shape=(jax.ShapeDtypeStruct((NPAD, H), jnp.bfloat16),
                   jax.ShapeDtypeStruct((NPAD, 8), jnp.float32)),
    )(x, W1, degp)


def _k23(a, y, dinv8, b, W):
    def body(a_ref, y_ref_in, dinv_ref, b_ref, w_ref, y_ref):
        dinv = dinv_ref[:, 0:1]
        s = (a_ref[0:NPAD, :].astype(jnp.float32)
             + a_ref[NPAD:2 * NPAD, :].astype(jnp.float32)
             + y_ref_in[...].astype(jnp.float32))
        h = jnp.maximum(dinv * s + b_ref[...], 0.0)
        y_ref[...] = (dinv * jnp.dot(
            h.astype(jnp.bfloat16), w_ref[...].astype(jnp.bfloat16),
            preferred_element_type=jnp.float32)).astype(jnp.bfloat16)

    return pl.pallas_call(
        body,
        out_shape=jax.ShapeDtypeStruct((NPAD, H), jnp.bfloat16),
    )(a, y, dinv8, b, W)


def _k4(a, y, dinv8, b3, batchp, Wf1, bf1, Wf2, bf2):
    def body(a_ref, y_ref, dinv_ref, b_ref, batch_ref, wf1_ref,
             bf1_ref, wf2_ref, bf2_ref, out_ref):
        dinv = dinv_ref[:, 0:1]
        s = (a_ref[0:NPAD, :].astype(jnp.float32)
             + a_ref[NPAD:2 * NPAD, :].astype(jnp.float32)
             + y_ref[...].astype(jnp.float32))
        h = jnp.maximum(dinv * s + b_ref[...], 0.0)
        gids = lax.broadcasted_iota(jnp.int32, (G, NPAD), 0)
        mt = (gids == batch_ref[...]).astype(jnp.bfloat16)
        sums = jnp.dot(mt, h.astype(jnp.bfloat16),
                       preferred_element_type=jnp.float32)
        counts = jnp.sum(mt.astype(jnp.float32), axis=1, keepdims=True)
        pooled = sums / jnp.maximum(counts, 1.0)
        hh = jnp.maximum(jnp.dot(pooled, wf1_ref[...], precision=_HI,
                                 preferred_element_type=jnp.float32)
                         + bf1_ref[...], 0.0)
        logits = jnp.dot(hh, wf2_ref[...], precision=_HI,
                         preferred_element_type=jnp.float32) + bf2_ref[...]
        ls = logits - jnp.max(logits, axis=1, keepdims=True)
        out_ref[...] = ls - jnp.log(jnp.sum(jnp.exp(ls), axis=1,
                                            keepdims=True))

    return pl.pallas_call(
        body,
        out_shape=jax.ShapeDtypeStruct((G, 2), jnp.float32),
    )(a, y, dinv8, b3, batchp, Wf1, bf1, Wf2, bf2)


def kernel(x, edge_index, batch, W1, b1, W2, b2, W3, b3, Wf1, bf1, Wf2, bf2):
    f32 = jnp.float32
    ei = edge_index.reshape(2, NTILES, NCH, CH)
    batchp = jnp.concatenate(
        [batch, jnp.full((NPAD - N,), -1, jnp.int32)]).reshape(1, NPAD)
    zeros64 = jnp.zeros((NPAD, H), jnp.bfloat16)
    zeros8 = jnp.zeros((NPAD, 8), jnp.bfloat16)
    ones8 = jnp.ones((CH, 8), jnp.bfloat16)

    degp = _deg_pass(ei, zeros8, ones8)
    y, dinv8 = _k1(x, W1, degp)
    a = _agg_pass(y, zeros64, ei)
    y = _k23(a, y, dinv8, b1.reshape(1, H), W2)
    a = _agg_pass(y, zeros64, ei)
    y = _k23(a, y, dinv8, b2.reshape(1, H), W3)
    a = _agg_pass(y, zeros64, ei)
    return _k4(a, y, dinv8, b3.reshape(1, H), batchp,
               Wf1, bf1.reshape(1, 32), Wf2, bf2.reshape(1, 2))
